# baseline (speedup 1.0000x reference)
import jax
import jax.numpy as jnp
from jax import lax
from jax.experimental import pallas as pl
from jax.experimental.pallas import tpu as pltpu
from jax.experimental.pallas import tpu_sc as plsc

B = 16384
DIM = 512
NBINS = 64
BOUND = 10.0
NLAYERS = 2
EPS = 1e-6

NC = 2
NS = 16
NW = NC * NS
CB = 4
RB = NW // CB
DPB = DIM // CB
LANES = 16
SUBS = DPB // LANES
ROWS_W = B // RB
NBCH = 64
NCH = ROWS_W // NBCH

NFIELD = 6
FSTR = DPB * NBINS
LSTR = NFIELD * FSTR
TABW = NLAYERS * LSTR

BBZ = 512


def _softplus(x):
    return jnp.maximum(x, 0.0) + jnp.log(1.0 + jnp.exp(-jnp.abs(x)))


def _prep_kernel(raw_wT, raw_hT, raw_sT, log_scale, TAB, SCALE):
    ir = lax.broadcasted_iota(jnp.int32, (NBINS, NBINS), 0)
    ic = lax.broadcasted_iota(jnp.int32, (NBINS, NBINS), 1)
    L_tri = (ic <= ir).astype(jnp.float32)

    def tables(raw):
        m = jnp.max(raw, axis=0, keepdims=True)
        e = jnp.exp(raw - m)
        w = e / jnp.sum(e, axis=0, keepdims=True) * (2.0 * BOUND)
        cw = lax.dot_general(L_tri, w, (((1,), (0,)), ((), ())),
                             preferred_element_type=jnp.float32)
        return w, cw

    neg = jnp.full((1, DIM), -BOUND, jnp.float32)
    for l in range(NLAYERS):
        w, cw = tables(raw_wT[l])
        h, ch = tables(raw_hT[l])
        s = _softplus(raw_sT[l]) + 1e-4
        fields = (
            jnp.concatenate([neg, cw[: NBINS - 1] - BOUND], axis=0),
            1.0 / w,
            jnp.concatenate([neg, ch[: NBINS - 1] - BOUND], axis=0),
            h,
            s[:NBINS],
            s[:NBINS] + s[1:] - 2.0 * (h / w),
        )
        for f, arr in enumerate(fields):
            for cb in range(CB):
                TAB[cb, l, f] = arr[:, cb * DPB:(cb + 1) * DPB]
    SCALE[...] = _softplus(log_scale[...]) + 1e-4


def _logit_kernel(u, z):
    u_safe = jnp.clip(u[...], EPS, 1.0 - EPS)
    z[...] = jnp.log(u_safe / (1.0 - u_safe))


def _rqs_group(zs, tab_v, dl):
    for l in range(NLAYERS):
        zcs = [jnp.minimum(jnp.maximum(z, -BOUND), BOUND) for z in zs]
        curs = [dl + (l * LSTR + sub * LANES) for sub in range(SUBS)]
        for step in (32, 16, 8, 4, 2, 1):
            cands = [c + step * DPB for c in curs]
            vals = [plsc.load_gather(tab_v, [cd]) for cd in cands]
            curs = [jnp.where(zc >= v, cd, c)
                    for zc, v, cd, c in zip(zcs, vals, cands, curs)]
        out = []
        for zc, cur, z in zip(zcs, curs, zs):
            xkb = plsc.load_gather(tab_v, [cur])
            invw = plsc.load_gather(tab_v, [cur + FSTR])
            yb = plsc.load_gather(tab_v, [cur + 2 * FSTR])
            h = plsc.load_gather(tab_v, [cur + 3 * FSTR])
            d0 = plsc.load_gather(tab_v, [cur + 4 * FSTR])
            e = plsc.load_gather(tab_v, [cur + 5 * FSTR])
            s = h * invw
            xi = jnp.minimum(jnp.maximum((zc - xkb) * invw, 0.0), 1.0)
            omxi = 1.0 - xi
            num = (h * xi) * (s * xi + d0 * omxi)
            den = s + e * (xi * omxi)
            y = yb + num / den
            inside = jnp.abs(z) < BOUND
            out.append(jnp.where(inside, y, z))
        zs = out
    return zs


def _sc_body(z_hbm, tau_hbm, tab_hbm, scale_hbm, bias_hbm, out_hbm,
             tab_v, zbuf, taubuf, sbbuf,
             isem0, isem1, isem2, osem0, osem1, osem2):
    w = lax.axis_index("s") * NC + lax.axis_index("c")
    cb = lax.rem(w, CB)
    rb = lax.div(w, CB)
    dcol = pl.multiple_of(cb * DPB, DPB)
    pltpu.sync_copy(tab_hbm.at[pl.ds(pl.multiple_of(cb * TABW, 8), TABW)], tab_v)
    pltpu.sync_copy(scale_hbm.at[pl.ds(dcol, DPB)], sbbuf.at[0])
    pltpu.sync_copy(bias_hbm.at[pl.ds(dcol, DPB)], sbbuf.at[1])
    dl = lax.iota(jnp.int32, LANES)
    rbase = rb * ROWS_W
    isems = (isem0, isem1, isem2)
    osems = (osem0, osem1, osem2)
    scalevs = [sbbuf[0, pl.ds(sub * LANES, LANES)] for sub in range(SUBS)]
    biasvs = [sbbuf[1, pl.ds(sub * LANES, LANES)] for sub in range(SUBS)]

    def in_copies(c, b):
        r0 = pl.multiple_of(rbase + c * NBCH, 8)
        return (
            pltpu.make_async_copy(
                z_hbm.at[pl.ds(r0, NBCH), pl.ds(dcol, DPB)], zbuf.at[b],
                isems[b]),
            pltpu.make_async_copy(tau_hbm.at[pl.ds(r0, NBCH)], taubuf.at[b],
                                  isems[b]),
        )

    def out_copy(c, b):
        r0 = pl.multiple_of(rbase + c * NBCH, 8)
        return pltpu.make_async_copy(
            zbuf.at[b], out_hbm.at[pl.ds(r0, NBCH), pl.ds(dcol, DPB)], osems[b])

    def do_chunk(c, b):
        for cp in in_copies(c, b):
            cp.wait()

        def row(q, carry, b=b):
            zs = [zbuf[b, q, pl.ds(sub * LANES, LANES)] for sub in range(SUBS)]
            zs = _rqs_group(zs, tab_v, dl)
            tauv = plsc.load_gather(taubuf.at[b],
                                    [jnp.full((LANES,), q, jnp.int32)])
            for sub in range(SUBS):
                zbuf[b, q, pl.ds(sub * LANES, LANES)] = (
                    tauv * (zs[sub] * scalevs[sub] + biasvs[sub]))
            return carry

        lax.fori_loop(0, NBCH, row, 0)
        out_copy(c, b).start()

    for cp in in_copies(0, 0):
        cp.start()
    for cp in in_copies(1, 1):
        cp.start()

    def outer(c3, carry):
        for k in range(3):
            c = c3 * 3 + k
            b = k
            b2 = (k + 2) % 3

            if k > 0:
                out_copy(0, b2).wait()
            else:
                @pl.when(c3 > 0)
                def _wait_prev_out(b2=b2):
                    out_copy(0, b2).wait()

            @pl.when(c + 2 < NCH)
            def _prefetch(c=c, b2=b2):
                for cp in in_copies(c + 2, b2):
                    cp.start()

            do_chunk(c, b)
        return carry

    lax.fori_loop(0, (NCH - 2) // 3, outer, 0)
    do_chunk(NCH - 2, (NCH - 2) % 3)
    do_chunk(NCH - 1, (NCH - 1) % 3)
    for c in (NCH - 3, NCH - 2, NCH - 1):
        out_copy(c, c % 3).wait()


@jax.jit
def kernel(u, tau, log_scale, bias, raw_w, raw_h, raw_s):
    TAB, SCALE = pl.pallas_call(
        _prep_kernel,
        out_shape=(jax.ShapeDtypeStruct((CB, NLAYERS, NFIELD, NBINS, DPB),
                                        jnp.float32),
                   jax.ShapeDtypeStruct((1, DIM), jnp.float32)),
    )(jnp.transpose(raw_w, (0, 2, 1)), jnp.transpose(raw_h, (0, 2, 1)),
      jnp.transpose(raw_s, (0, 2, 1)), log_scale.reshape(1, DIM))

    z = pl.pallas_call(
        _logit_kernel,
        grid=(B // BBZ,),
        in_specs=[pl.BlockSpec((BBZ, DIM), lambda i: (i, 0))],
        out_specs=pl.BlockSpec((BBZ, DIM), lambda i: (i, 0)),
        out_shape=jax.ShapeDtypeStruct((B, DIM), jnp.float32),
    )(u)

    mesh = plsc.VectorSubcoreMesh(core_axis_name="c", subcore_axis_name="s",
                                  num_cores=NC, num_subcores=NS)
    out = pl.kernel(
        _sc_body,
        out_type=jax.ShapeDtypeStruct((B, DIM), jnp.float32),
        mesh=mesh,
        compiler_params=pltpu.CompilerParams(needs_layout_passes=False),
        scratch_types=[
            pltpu.VMEM((TABW,), jnp.float32),
            pltpu.VMEM((3, NBCH, DPB), jnp.float32),
            pltpu.VMEM((3, NBCH), jnp.float32),
            pltpu.VMEM((2, DPB), jnp.float32),
            pltpu.SemaphoreType.DMA,
            pltpu.SemaphoreType.DMA,
            pltpu.SemaphoreType.DMA,
            pltpu.SemaphoreType.DMA,
            pltpu.SemaphoreType.DMA,
            pltpu.SemaphoreType.DMA,
        ],
    )(z, tau.reshape(B), TAB.reshape(CB * TABW), SCALE.reshape(DIM), bias)
    return out


# speedup vs baseline: 1.0109x; 1.0109x over previous
import jax
import jax.numpy as jnp
from jax import lax
from jax.experimental import pallas as pl
from jax.experimental.pallas import tpu as pltpu
from jax.experimental.pallas import tpu_sc as plsc

B = 16384
DIM = 512
NBINS = 64
BOUND = 10.0
NLAYERS = 2
EPS = 1e-6

NC = 2
NS = 16
NW = NC * NS
CB = 4
RB = NW // CB
DPB = DIM // CB
LANES = 16
SUBS = DPB // LANES
ROWS_W = B // RB
NBCH = 64
NCH = ROWS_W // NBCH

NFIELD = 6
FSTR = DPB * NBINS
LSTR = NFIELD * FSTR
TABW = NLAYERS * LSTR

BBZ = 512


def _softplus(x):
    return jnp.maximum(x, 0.0) + jnp.log(1.0 + jnp.exp(-jnp.abs(x)))


def _prep_kernel(raw_wT, raw_hT, raw_sT, log_scale, TAB, SCALE):
    ir = lax.broadcasted_iota(jnp.int32, (NBINS, NBINS), 0)
    ic = lax.broadcasted_iota(jnp.int32, (NBINS, NBINS), 1)
    L_tri = (ic <= ir).astype(jnp.float32)

    def tables(raw):
        m = jnp.max(raw, axis=0, keepdims=True)
        e = jnp.exp(raw - m)
        w = e / jnp.sum(e, axis=0, keepdims=True) * (2.0 * BOUND)
        cw = lax.dot_general(L_tri, w, (((1,), (0,)), ((), ())),
                             preferred_element_type=jnp.float32)
        return w, cw

    neg = jnp.full((1, DIM), -BOUND, jnp.float32)
    for l in range(NLAYERS):
        w, cw = tables(raw_wT[l])
        h, ch = tables(raw_hT[l])
        s = _softplus(raw_sT[l]) + 1e-4
        fields = (
            jnp.concatenate([neg, cw[: NBINS - 1] - BOUND], axis=0),
            1.0 / w,
            jnp.concatenate([neg, ch[: NBINS - 1] - BOUND], axis=0),
            h,
            s[:NBINS],
            s[:NBINS] + s[1:] - 2.0 * (h / w),
        )
        for f, arr in enumerate(fields):
            for cb in range(CB):
                TAB[cb, l, f] = arr[:, cb * DPB:(cb + 1) * DPB]
    SCALE[...] = _softplus(log_scale[...]) + 1e-4


def _logit_kernel(u, z):
    u_safe = jnp.clip(u[...], EPS, 1.0 - EPS)
    z[...] = jnp.log(u_safe / (1.0 - u_safe))


def _rqs_group(zs, tab_v, dl):
    for l in range(NLAYERS):
        zcs = [jnp.minimum(jnp.maximum(z, -BOUND), BOUND) for z in zs]
        curs = [dl + (l * LSTR + sub * LANES) for sub in range(SUBS)]
        for step in (32, 16, 8, 4, 2, 1):
            cands = [c + step * DPB for c in curs]
            vals = [plsc.load_gather(tab_v, [cd]) for cd in cands]
            curs = [jnp.where(zc >= v, cd, c)
                    for zc, v, cd, c in zip(zcs, vals, cands, curs)]
        out = []
        for zc, cur, z in zip(zcs, curs, zs):
            xkb = plsc.load_gather(tab_v, [cur])
            invw = plsc.load_gather(tab_v, [cur + FSTR])
            yb = plsc.load_gather(tab_v, [cur + 2 * FSTR])
            h = plsc.load_gather(tab_v, [cur + 3 * FSTR])
            d0 = plsc.load_gather(tab_v, [cur + 4 * FSTR])
            e = plsc.load_gather(tab_v, [cur + 5 * FSTR])
            s = h * invw
            xi = jnp.minimum(jnp.maximum((zc - xkb) * invw, 0.0), 1.0)
            omxi = 1.0 - xi
            num = (h * xi) * (s * xi + d0 * omxi)
            den = s + e * (xi * omxi)
            y = yb + num / den
            inside = jnp.abs(z) < BOUND
            out.append(jnp.where(inside, y, z))
        zs = out
    return zs


def _sc_body(z_hbm, tau_hbm, tab_hbm, scale_hbm, bias_hbm, out_hbm,
             tab_v, zbuf, taubuf, sbbuf, sem0, sem1):
    w = lax.axis_index("s") * NC + lax.axis_index("c")
    cb = lax.rem(w, CB)
    rb = lax.div(w, CB)
    dcol = pl.multiple_of(cb * DPB, DPB)
    pltpu.sync_copy(tab_hbm.at[pl.ds(pl.multiple_of(cb * TABW, 8), TABW)], tab_v)
    pltpu.sync_copy(scale_hbm.at[pl.ds(dcol, DPB)], sbbuf.at[0])
    pltpu.sync_copy(bias_hbm.at[pl.ds(dcol, DPB)], sbbuf.at[1])
    dl = lax.iota(jnp.int32, LANES)
    rbase = rb * ROWS_W
    sems = (sem0, sem1)
    scalevs = [sbbuf[0, pl.ds(sub * LANES, LANES)] for sub in range(SUBS)]
    biasvs = [sbbuf[1, pl.ds(sub * LANES, LANES)] for sub in range(SUBS)]

    def in_copies(c, b):
        r0 = pl.multiple_of(rbase + c * NBCH, 8)
        return (
            pltpu.make_async_copy(
                z_hbm.at[pl.ds(r0, NBCH), pl.ds(dcol, DPB)], zbuf.at[b], sems[b]),
            pltpu.make_async_copy(tau_hbm.at[pl.ds(r0, NBCH)], taubuf.at[b],
                                  sems[b]),
        )

    for cp in in_copies(0, 0):
        cp.start()

    def outer(c2, carry):
        for b in range(2):
            c = c2 * 2 + b
            r0 = pl.multiple_of(rbase + c * NBCH, 8)

            @pl.when(c + 1 < NCH)
            def _prefetch(c=c, b=b):
                for cp in in_copies(c + 1, 1 - b):
                    cp.start()

            for cp in in_copies(c, b):
                cp.wait()

            def row(q, carry, b=b):
                zs = [zbuf[b, q, pl.ds(sub * LANES, LANES)]
                      for sub in range(SUBS)]
                zs = _rqs_group(zs, tab_v, dl)
                tauv = plsc.load_gather(taubuf.at[b],
                                        [jnp.full((LANES,), q, jnp.int32)])
                for sub in range(SUBS):
                    zbuf[b, q, pl.ds(sub * LANES, LANES)] = (
                        tauv * (zs[sub] * scalevs[sub] + biasvs[sub]))
                return carry

            lax.fori_loop(0, NBCH, row, 0)
            pltpu.sync_copy(zbuf.at[b],
                            out_hbm.at[pl.ds(r0, NBCH), pl.ds(dcol, DPB)])
        return carry

    lax.fori_loop(0, NCH // 2, outer, 0)


@jax.jit
def kernel(u, tau, log_scale, bias, raw_w, raw_h, raw_s):
    TAB, SCALE = pl.pallas_call(
        _prep_kernel,
        out_shape=(jax.ShapeDtypeStruct((CB, NLAYERS, NFIELD, NBINS, DPB),
                                        jnp.float32),
                   jax.ShapeDtypeStruct((1, DIM), jnp.float32)),
    )(jnp.transpose(raw_w, (0, 2, 1)), jnp.transpose(raw_h, (0, 2, 1)),
      jnp.transpose(raw_s, (0, 2, 1)), log_scale.reshape(1, DIM))

    z = pl.pallas_call(
        _logit_kernel,
        grid=(B // BBZ,),
        in_specs=[pl.BlockSpec((BBZ, DIM), lambda i: (i, 0))],
        out_specs=pl.BlockSpec((BBZ, DIM), lambda i: (i, 0)),
        out_shape=jax.ShapeDtypeStruct((B, DIM), jnp.float32),
    )(u)

    mesh = plsc.VectorSubcoreMesh(core_axis_name="c", subcore_axis_name="s",
                                  num_cores=NC, num_subcores=NS)
    out = pl.kernel(
        _sc_body,
        out_type=jax.ShapeDtypeStruct((B, DIM), jnp.float32),
        mesh=mesh,
        compiler_params=pltpu.CompilerParams(needs_layout_passes=False),
        scratch_types=[
            pltpu.VMEM((TABW,), jnp.float32),
            pltpu.VMEM((2, NBCH, DPB), jnp.float32),
            pltpu.VMEM((2, NBCH), jnp.float32),
            pltpu.VMEM((2, DPB), jnp.float32),
            pltpu.SemaphoreType.DMA,
            pltpu.SemaphoreType.DMA,
        ],
    )(z, tau.reshape(B), TAB.reshape(CB * TABW), SCALE.reshape(DIM), bias)
    return out
